# Initial kernel scaffold; baseline (speedup 1.0000x reference)
import functools

import jax
import jax.numpy as jnp
from jax import lax
from jax.experimental import pallas as pl
from jax.experimental.pallas import tpu as pltpu
from jax.experimental.pallas import tpu_sc as plsc

_N = 10000
_E = 320000
_D = 128
_H = 128
_G = 64
_AUGW = 144
_CHUNK = 128
_NCHUNKS = _E // _CHUNK
_NW = 32
_ROWS_PER_TILE = _N // 16
_BLK = 1000
_NBLK = _N // _BLK


def _sc_edge_agg_body(yl_hbm, src_hbm, dst_hbm, zeros_hbm, out_hbm,
                      src_v, dst_v, rows_v, agg_sh, sem):
    c = lax.axis_index("c")
    s = lax.axis_index("s")
    wid = s * 2 + c
    pltpu.sync_copy(zeros_hbm.at[pl.ds(s * _ROWS_PER_TILE, _ROWS_PER_TILE)],
                    agg_sh.at[pl.ds(s * _ROWS_PER_TILE, _ROWS_PER_TILE)])
    plsc.subcore_barrier()

    nfull = _NCHUNKS // _NW
    nrem = _NCHUNKS - nfull * _NW
    nw = nfull + jnp.where(wid < nrem, 1, 0)

    def body(j, carry):
        base = (wid + j * _NW) * _CHUNK
        pltpu.sync_copy(src_hbm.at[pl.ds(base, _CHUNK)], src_v)
        pltpu.sync_copy(dst_hbm.at[pl.ds(base, _CHUNK)], dst_v)
        pltpu.async_copy(yl_hbm.at[src_v], rows_v, sem).wait()
        pltpu.sync_copy(rows_v, agg_sh.at[dst_v], add=True)
        return carry

    lax.fori_loop(0, nw, body, 0)
    plsc.subcore_barrier()
    pltpu.sync_copy(agg_sh.at[pl.ds(s * _ROWS_PER_TILE, _ROWS_PER_TILE)],
                    out_hbm.at[c, pl.ds(s * _ROWS_PER_TILE, _ROWS_PER_TILE)])


def _edge_agg(yl_aug, src, dst, zeros):
    call = pl.kernel(
        _sc_edge_agg_body,
        out_type=jax.ShapeDtypeStruct((2, _N, _AUGW), jnp.float32),
        mesh=plsc.VectorSubcoreMesh(core_axis_name="c", subcore_axis_name="s"),
        scratch_types=[
            pltpu.VMEM((_CHUNK,), jnp.int32),
            pltpu.VMEM((_CHUNK,), jnp.int32),
            pltpu.VMEM((_CHUNK, _AUGW), jnp.float32),
            pltpu.VMEM_SHARED((_N, _AUGW), jnp.float32),
            pltpu.SemaphoreType.DMA,
        ],
    )
    return call(yl_aug, src, dst, zeros)


def _k1_body(nb, eb, wla, wr, w1, w2, bcol, blr, b1r, b2r,
             yl_out, base_out, mlp_out):
    xb = jnp.concatenate([nb[...], eb[...]], axis=1)
    yl_out[...] = jnp.dot(xb, wla[...], preferred_element_type=jnp.float32) + bcol[...]
    base_out[...] = jnp.dot(xb, wr[...], preferred_element_type=jnp.float32) + blr[...]
    h1 = jnp.maximum(jnp.dot(xb, w1[...], preferred_element_type=jnp.float32) + b1r[...], 0.0)
    mlp_out[...] = jnp.maximum(jnp.dot(h1, w2[...], preferred_element_type=jnp.float32) + b2r[...], 0.0)


def _k1(nf, ef, wla, wr, w1, w2, bcol, blr, b1r, b2r):
    return pl.pallas_call(
        _k1_body,
        grid=(_NBLK,),
        in_specs=[
            pl.BlockSpec((_BLK, _D), lambda i: (i, 0)),
            pl.BlockSpec((_BLK, _D), lambda i: (i, 0)),
            pl.BlockSpec((2 * _D, _AUGW), lambda i: (0, 0)),
            pl.BlockSpec((2 * _D, _H), lambda i: (0, 0)),
            pl.BlockSpec((2 * _D, _H), lambda i: (0, 0)),
            pl.BlockSpec((_H, _H), lambda i: (0, 0)),
            pl.BlockSpec((1, _AUGW), lambda i: (0, 0)),
            pl.BlockSpec((1, _H), lambda i: (0, 0)),
            pl.BlockSpec((1, _H), lambda i: (0, 0)),
            pl.BlockSpec((1, _H), lambda i: (0, 0)),
        ],
        out_specs=[
            pl.BlockSpec((_BLK, _AUGW), lambda i: (i, 0)),
            pl.BlockSpec((_BLK, _H), lambda i: (i, 0)),
            pl.BlockSpec((_BLK, _H), lambda i: (i, 0)),
        ],
        out_shape=[
            jax.ShapeDtypeStruct((_N, _AUGW), jnp.float32),
            jax.ShapeDtypeStruct((_N, _H), jnp.float32),
            jax.ShapeDtypeStruct((_N, _H), jnp.float32),
        ],
    )(nf, ef, wla, wr, w1, w2, bcol, blr, b1r, b2r)


def _k3_body(a0, a1, baseb, pre_out, sums, sumsq):
    i = pl.program_id(0)
    aggb = a0[...] + a1[...]
    deg = jnp.maximum(aggb[:, _H:_H + 1], 1.0)
    pre = aggb[:, :_H] / deg + baseb[...]
    pre_out[...] = pre

    @pl.when(i == 0)
    def _():
        sums[...] = jnp.zeros_like(sums)
        sumsq[...] = jnp.zeros_like(sumsq)

    sums[...] += jnp.sum(pre, axis=0, keepdims=True)
    sumsq[...] += jnp.sum(pre * pre, axis=0, keepdims=True)


def _k3(a0, a1, base):
    return pl.pallas_call(
        _k3_body,
        grid=(_NBLK,),
        in_specs=[
            pl.BlockSpec((_BLK, _AUGW), lambda i: (i, 0)),
            pl.BlockSpec((_BLK, _AUGW), lambda i: (i, 0)),
            pl.BlockSpec((_BLK, _H), lambda i: (i, 0)),
        ],
        out_specs=[
            pl.BlockSpec((_BLK, _H), lambda i: (i, 0)),
            pl.BlockSpec((1, _H), lambda i: (0, 0)),
            pl.BlockSpec((1, _H), lambda i: (0, 0)),
        ],
        out_shape=[
            jax.ShapeDtypeStruct((_N, _H), jnp.float32),
            jax.ShapeDtypeStruct((1, _H), jnp.float32),
            jax.ShapeDtypeStruct((1, _H), jnp.float32),
        ],
    )(a0, a1, base)


def _k4_body(preb, mlpb, batchb, sums, sumsq, gam, bet, wf, bfr,
             out, gacc, cacc):
    i = pl.program_id(0)
    mu = sums[...] / _N
    var = sumsq[...] / _N - mu * mu
    rstd = lax.rsqrt(var + 1e-5)
    xg = (preb[...] - mu) * rstd * gam[...] + bet[...]
    comb = jnp.maximum(xg, 0.0) + mlpb[...]
    b = batchb[0]
    gi = lax.broadcasted_iota(jnp.int32, (_G, 1), 0)
    oh = (gi == b).astype(jnp.float32)

    @pl.when(i == 0)
    def _():
        gacc[...] = jnp.zeros_like(gacc)
        cacc[...] = jnp.zeros_like(cacc)

    gacc[...] += jnp.dot(oh, comb, preferred_element_type=jnp.float32)
    cacc[...] += jnp.sum(oh, axis=1, keepdims=True)

    @pl.when(i == pl.num_programs(0) - 1)
    def _():
        gf = gacc[...] / jnp.maximum(cacc[...], 1.0)
        z = jnp.dot(gf, wf[...], preferred_element_type=jnp.float32) + bfr[...]
        out[...] = jax.nn.sigmoid(z)


def _k4(pre, mlp, batch3, sums, sumsq, gam, bet, wf, bfr):
    return pl.pallas_call(
        _k4_body,
        grid=(_NBLK,),
        in_specs=[
            pl.BlockSpec((_BLK, _H), lambda i: (i, 0)),
            pl.BlockSpec((_BLK, _H), lambda i: (i, 0)),
            pl.BlockSpec((1, 1, _BLK), lambda i: (i, 0, 0)),
            pl.BlockSpec((1, _H), lambda i: (0, 0)),
            pl.BlockSpec((1, _H), lambda i: (0, 0)),
            pl.BlockSpec((1, _H), lambda i: (0, 0)),
            pl.BlockSpec((1, _H), lambda i: (0, 0)),
            pl.BlockSpec((_H, 1), lambda i: (0, 0)),
            pl.BlockSpec((1, 1), lambda i: (0, 0)),
        ],
        out_specs=pl.BlockSpec((_G, 1), lambda i: (0, 0)),
        out_shape=jax.ShapeDtypeStruct((_G, 1), jnp.float32),
        scratch_shapes=[
            pltpu.VMEM((_G, _H), jnp.float32),
            pltpu.VMEM((_G, 1), jnp.float32),
        ],
    )(pre, mlp, batch3, sums, sumsq, gam, bet, wf, bfr)


def kernel(normal_features, extreme_features, Wl, bl, Wr, gamma, beta,
           W1, b1, W2, b2, Wf, bf, edge_index, batch):
    f32 = jnp.float32
    wla = jnp.concatenate([Wl, jnp.zeros((2 * _D, _AUGW - _H), f32)], axis=1)
    bcol = jnp.zeros((1, _AUGW), f32).at[0, _H].set(1.0)
    blr = bl.reshape(1, _H)
    b1r = b1.reshape(1, _H)
    b2r = b2.reshape(1, _H)
    bfr = bf.reshape(1, 1)
    gam = gamma.reshape(1, _H)
    bet = beta.reshape(1, _H)

    yl_aug, base, mlp = _k1(normal_features, extreme_features,
                            wla, Wr, W1, W2, bcol, blr, b1r, b2r)

    zeros = jnp.zeros((_N, _AUGW), f32)
    agg2 = _edge_agg(yl_aug, edge_index[0], edge_index[1], zeros)

    pre, sums, sumsq = _k3(agg2[0], agg2[1], base)

    batch3 = batch.reshape(_NBLK, 1, _BLK)
    return _k4(pre, mlp, batch3, sums, sumsq, gam, bet, Wf, bfr)


# speedup vs baseline: 7.7206x; 7.7206x over previous
import functools

import jax
import jax.numpy as jnp
from jax import lax
from jax.experimental import pallas as pl
from jax.experimental.pallas import tpu as pltpu
from jax.experimental.pallas import tpu_sc as plsc

_N = 10000
_E = 320000
_D = 128
_H = 128
_G = 64
_AUGW = 144
_CHUNK = 128
_NCHUNKS = _E // _CHUNK
_NW = 32
_NPAD = 10240
_ROWS_PER_TILE = _NPAD // 16
_BLK = 1000
_NBLK = _N // _BLK


def _sc_edge_agg_body(yl_hbm, src_hbm, dst_hbm, zeros_hbm, out_hbm,
                      src_v, dst_v, rows_v, agg_sh, sem):
    c = lax.axis_index("c")
    s = lax.axis_index("s")
    wid = s * 2 + c
    pltpu.sync_copy(zeros_hbm.at[pl.ds(s * _ROWS_PER_TILE, _ROWS_PER_TILE)],
                    agg_sh.at[pl.ds(s * _ROWS_PER_TILE, _ROWS_PER_TILE)])
    plsc.subcore_barrier()

    nfull = _NCHUNKS // _NW
    nrem = _NCHUNKS - nfull * _NW
    nw = nfull + jnp.where(wid < nrem, 1, 0)

    def body(j, carry):
        base = (wid + j * _NW) * _CHUNK
        pltpu.sync_copy(src_hbm.at[pl.ds(base, _CHUNK)], src_v)
        pltpu.sync_copy(dst_hbm.at[pl.ds(base, _CHUNK)], dst_v)
        pltpu.async_copy(yl_hbm.at[src_v], rows_v, sem).wait()
        pltpu.sync_copy(rows_v, agg_sh.at[dst_v], add=True)
        return carry

    lax.fori_loop(0, nw, body, 0)
    plsc.subcore_barrier()
    pltpu.sync_copy(agg_sh.at[pl.ds(s * _ROWS_PER_TILE, _ROWS_PER_TILE)],
                    out_hbm.at[c, pl.ds(s * _ROWS_PER_TILE, _ROWS_PER_TILE)])


def _edge_agg(yl_aug, src, dst, zeros):
    call = pl.kernel(
        _sc_edge_agg_body,
        out_type=jax.ShapeDtypeStruct((2, _NPAD, _AUGW), jnp.float32),
        mesh=plsc.VectorSubcoreMesh(core_axis_name="c", subcore_axis_name="s"),
        scratch_types=[
            pltpu.VMEM((_CHUNK,), jnp.int32),
            pltpu.VMEM((_CHUNK,), jnp.int32),
            pltpu.VMEM((_CHUNK, _AUGW), jnp.float32),
            pltpu.VMEM_SHARED((_NPAD, _AUGW), jnp.float32),
            pltpu.SemaphoreType.DMA,
        ],
        compiler_params=pltpu.CompilerParams(use_tc_tiling_on_sc=False),
    )
    return call(yl_aug, src, dst, zeros)


def _k1_body(nb, eb, wla, wr, w1, w2, bcol, blr, b1r, b2r,
             yl_out, base_out, mlp_out):
    xb = jnp.concatenate([nb[...], eb[...]], axis=1)
    yl_out[...] = jnp.dot(xb, wla[...], preferred_element_type=jnp.float32) + bcol[...]
    base_out[...] = jnp.dot(xb, wr[...], preferred_element_type=jnp.float32) + blr[...]
    h1 = jnp.maximum(jnp.dot(xb, w1[...], preferred_element_type=jnp.float32) + b1r[...], 0.0)
    mlp_out[...] = jnp.maximum(jnp.dot(h1, w2[...], preferred_element_type=jnp.float32) + b2r[...], 0.0)


def _k1(nf, ef, wla, wr, w1, w2, bcol, blr, b1r, b2r):
    return pl.pallas_call(
        _k1_body,
        grid=(_NBLK,),
        in_specs=[
            pl.BlockSpec((_BLK, _D), lambda i: (i, 0)),
            pl.BlockSpec((_BLK, _D), lambda i: (i, 0)),
            pl.BlockSpec((2 * _D, _AUGW), lambda i: (0, 0)),
            pl.BlockSpec((2 * _D, _H), lambda i: (0, 0)),
            pl.BlockSpec((2 * _D, _H), lambda i: (0, 0)),
            pl.BlockSpec((_H, _H), lambda i: (0, 0)),
            pl.BlockSpec((1, _AUGW), lambda i: (0, 0)),
            pl.BlockSpec((1, _H), lambda i: (0, 0)),
            pl.BlockSpec((1, _H), lambda i: (0, 0)),
            pl.BlockSpec((1, _H), lambda i: (0, 0)),
        ],
        out_specs=[
            pl.BlockSpec((_BLK, _AUGW), lambda i: (i, 0)),
            pl.BlockSpec((_BLK, _H), lambda i: (i, 0)),
            pl.BlockSpec((_BLK, _H), lambda i: (i, 0)),
        ],
        out_shape=[
            jax.ShapeDtypeStruct((_N, _AUGW), jnp.float32),
            jax.ShapeDtypeStruct((_N, _H), jnp.float32),
            jax.ShapeDtypeStruct((_N, _H), jnp.float32),
        ],
    )(nf, ef, wla, wr, w1, w2, bcol, blr, b1r, b2r)


def _k3_body(a0, a1, baseb, pre_out, sums, sumsq):
    i = pl.program_id(0)
    aggb = a0[...] + a1[...]
    deg = jnp.maximum(aggb[:, _H:_H + 1], 1.0)
    pre = aggb[:, :_H] / deg + baseb[...]
    pre_out[...] = pre

    @pl.when(i == 0)
    def _():
        sums[...] = jnp.zeros_like(sums)
        sumsq[...] = jnp.zeros_like(sumsq)

    sums[...] += jnp.sum(pre, axis=0, keepdims=True)
    sumsq[...] += jnp.sum(pre * pre, axis=0, keepdims=True)


def _k3(a0, a1, base):
    return pl.pallas_call(
        _k3_body,
        grid=(_NBLK,),
        in_specs=[
            pl.BlockSpec((_BLK, _AUGW), lambda i: (i, 0)),
            pl.BlockSpec((_BLK, _AUGW), lambda i: (i, 0)),
            pl.BlockSpec((_BLK, _H), lambda i: (i, 0)),
        ],
        out_specs=[
            pl.BlockSpec((_BLK, _H), lambda i: (i, 0)),
            pl.BlockSpec((1, _H), lambda i: (0, 0)),
            pl.BlockSpec((1, _H), lambda i: (0, 0)),
        ],
        out_shape=[
            jax.ShapeDtypeStruct((_N, _H), jnp.float32),
            jax.ShapeDtypeStruct((1, _H), jnp.float32),
            jax.ShapeDtypeStruct((1, _H), jnp.float32),
        ],
    )(a0, a1, base)


def _k4_body(preb, mlpb, batchb, sums, sumsq, gam, bet, wf, bfr,
             out, gacc, cacc):
    i = pl.program_id(0)
    mu = sums[...] / _N
    var = sumsq[...] / _N - mu * mu
    rstd = lax.rsqrt(var + 1e-5)
    xg = (preb[...] - mu) * rstd * gam[...] + bet[...]
    comb = jnp.maximum(xg, 0.0) + mlpb[...]
    b = batchb[0]
    gi = lax.broadcasted_iota(jnp.int32, (_G, 1), 0)
    oh = (gi == b).astype(jnp.float32)

    @pl.when(i == 0)
    def _():
        gacc[...] = jnp.zeros_like(gacc)
        cacc[...] = jnp.zeros_like(cacc)

    gacc[...] += jnp.dot(oh, comb, preferred_element_type=jnp.float32)
    cacc[...] += jnp.sum(oh, axis=1, keepdims=True)

    @pl.when(i == pl.num_programs(0) - 1)
    def _():
        gf = gacc[...] / jnp.maximum(cacc[...], 1.0)
        z = jnp.dot(gf, wf[...], preferred_element_type=jnp.float32) + bfr[...]
        out[...] = jax.nn.sigmoid(z)


def _k4(pre, mlp, batch3, sums, sumsq, gam, bet, wf, bfr):
    return pl.pallas_call(
        _k4_body,
        grid=(_NBLK,),
        in_specs=[
            pl.BlockSpec((_BLK, _H), lambda i: (i, 0)),
            pl.BlockSpec((_BLK, _H), lambda i: (i, 0)),
            pl.BlockSpec((1, 1, _BLK), lambda i: (i, 0, 0)),
            pl.BlockSpec((1, _H), lambda i: (0, 0)),
            pl.BlockSpec((1, _H), lambda i: (0, 0)),
            pl.BlockSpec((1, _H), lambda i: (0, 0)),
            pl.BlockSpec((1, _H), lambda i: (0, 0)),
            pl.BlockSpec((_H, 1), lambda i: (0, 0)),
            pl.BlockSpec((1, 1), lambda i: (0, 0)),
        ],
        out_specs=pl.BlockSpec((_G, 1), lambda i: (0, 0)),
        out_shape=jax.ShapeDtypeStruct((_G, 1), jnp.float32),
        scratch_shapes=[
            pltpu.VMEM((_G, _H), jnp.float32),
            pltpu.VMEM((_G, 1), jnp.float32),
        ],
    )(pre, mlp, batch3, sums, sumsq, gam, bet, wf, bfr)


def kernel(normal_features, extreme_features, Wl, bl, Wr, gamma, beta,
           W1, b1, W2, b2, Wf, bf, edge_index, batch):
    f32 = jnp.float32
    wla = jnp.concatenate([Wl, jnp.zeros((2 * _D, _AUGW - _H), f32)], axis=1)
    bcol = jnp.zeros((1, _AUGW), f32).at[0, _H].set(1.0)
    blr = bl.reshape(1, _H)
    b1r = b1.reshape(1, _H)
    b2r = b2.reshape(1, _H)
    bfr = bf.reshape(1, 1)
    gam = gamma.reshape(1, _H)
    bet = beta.reshape(1, _H)

    yl_aug, base, mlp = _k1(normal_features, extreme_features,
                            wla, Wr, W1, W2, bcol, blr, b1r, b2r)

    zeros = jnp.zeros((_NPAD, _AUGW), f32)
    agg2 = _edge_agg(yl_aug, edge_index[0], edge_index[1], zeros)

    pre, sums, sumsq = _k3(agg2[0, :_N], agg2[1, :_N], base)

    batch3 = batch.reshape(_NBLK, 1, _BLK)
    return _k4(pre, mlp, batch3, sums, sumsq, gam, bet, Wf, bfr)
